# baseline (speedup 1.0000x reference)
import jax
import jax.numpy as jnp
from jax.experimental import pallas as pl

_NUM_MODELS = 64
_BLK = 2048


def _mlp_body(x_ref, w0_ref, b0_ref, w1_ref, b1_ref, w2_ref, b2_ref,
              w3_ref, b3_ref, w4_ref, b4_ref,
              out_ref, idx_ref, logit_ref, prob_ref):
    x = x_ref[...]
    h = jnp.dot(x, w0_ref[0], preferred_element_type=jnp.float32) + b0_ref[...]
    h = jnp.maximum(h, 0.0)
    h = jnp.dot(h, w1_ref[0], preferred_element_type=jnp.float32) + b1_ref[...]
    h = jnp.maximum(h, 0.0)
    h = jnp.dot(h, w2_ref[0], preferred_element_type=jnp.float32) + b2_ref[0]
    h = jnp.maximum(h, 0.0)
    h = jnp.dot(h, w3_ref[0], preferred_element_type=jnp.float32) + b3_ref[0]
    h = jnp.maximum(h, 0.0)
    y = jnp.dot(h, w4_ref[0], preferred_element_type=jnp.float32) + b4_ref[0]
    n_rep = out_ref.shape[1] // y.shape[1]
    out_ref[...] = jnp.concatenate([y] * n_rep, axis=1)
    idx_ref[...] = jnp.zeros(idx_ref.shape, jnp.int32)
    col = jax.lax.broadcasted_iota(jnp.int32, logit_ref.shape, 1)
    logit_ref[...] = jnp.where(col == 0, 0.0, -999.9).astype(jnp.float32)
    prob_ref[...] = jnp.where(col == 0, 1.0, 0.0).astype(jnp.float32)


def kernel(inputs, lods, W0, b0, W1, b1, W2, b2, W3, b3, W4, b4):
    bsz, in_f = inputs.shape
    hid = W1.shape[-1]
    out_f = W4.shape[-1]
    n_lods = int(lods.shape[0])
    grid = (bsz // _BLK,)

    full = lambda shape: pl.BlockSpec(shape, lambda i, _n=None: (0,) * len(shape))
    out9, idx, logits, probs = pl.pallas_call(
        _mlp_body,
        grid=grid,
        in_specs=[
            pl.BlockSpec((_BLK, in_f), lambda i: (i, 0)),
            full((1, in_f, hid)), full((1, hid)),
            full((1, hid, hid)), full((1, hid)),
            full((1, hid, hid)), full((1, 1, hid)),
            full((1, hid, hid)), full((1, 1, hid)),
            full((1, hid, out_f)), full((1, 1, out_f)),
        ],
        out_specs=[
            pl.BlockSpec((_BLK, n_lods * out_f), lambda i: (i, 0)),
            pl.BlockSpec((_BLK, 1), lambda i: (i, 0)),
            pl.BlockSpec((_BLK, _NUM_MODELS), lambda i: (i, 0)),
            pl.BlockSpec((_BLK, _NUM_MODELS), lambda i: (i, 0)),
        ],
        out_shape=[
            jax.ShapeDtypeStruct((bsz, n_lods * out_f), jnp.float32),
            jax.ShapeDtypeStruct((bsz, 1), jnp.int32),
            jax.ShapeDtypeStruct((bsz, _NUM_MODELS), jnp.float32),
            jax.ShapeDtypeStruct((bsz, _NUM_MODELS), jnp.float32),
        ],
    )(inputs, W0, b0, W1, b1,
      W2, b2.reshape(b2.shape[0], 1, hid),
      W3, b3.reshape(b3.shape[0], 1, hid),
      W4, b4.reshape(b4.shape[0], 1, out_f))

    model_outputs = out9.reshape(bsz, n_lods, out_f)
    return (model_outputs, idx.reshape(bsz), logits, probs)


# speedup vs baseline: 2.2697x; 2.2697x over previous
import jax
import jax.numpy as jnp
from jax.experimental import pallas as pl

_NUM_MODELS = 64
_BLK = 8192


def _mlp_body(x_ref, w0_ref, b0_ref, w1_ref, b1_ref, w2_ref, b2_ref,
              w3_ref, b3_ref, w4_ref, b4_ref,
              y_ref, logit_ref, prob_ref):
    x = x_ref[...]
    h = jnp.dot(w0_ref[...], x, preferred_element_type=jnp.float32) + b0_ref[...]
    h = jnp.maximum(h, 0.0)
    h = jnp.dot(w1_ref[...], h, preferred_element_type=jnp.float32) + b1_ref[...]
    h = jnp.maximum(h, 0.0)
    h = jnp.dot(w2_ref[...], h, preferred_element_type=jnp.float32) + b2_ref[...]
    h = jnp.maximum(h, 0.0)
    h = jnp.dot(w3_ref[...], h, preferred_element_type=jnp.float32) + b3_ref[...]
    h = jnp.maximum(h, 0.0)
    y_ref[...] = (jnp.dot(w4_ref[...], h, preferred_element_type=jnp.float32)
                  + b4_ref[...])
    col = jax.lax.broadcasted_iota(jnp.int32, logit_ref.shape, 1)
    logit_ref[...] = jnp.where(col == 0, 0.0, -999.9).astype(jnp.float32)
    prob_ref[...] = jnp.where(col == 0, 1.0, 0.0).astype(jnp.float32)


def kernel(inputs, lods, W0, b0, W1, b1, W2, b2, W3, b3, W4, b4):
    bsz, in_f = inputs.shape
    hid = W1.shape[-1]
    out_f = W4.shape[-1]
    n_lods = int(lods.shape[0])
    grid = (bsz // _BLK,)

    xT = inputs.T
    w0t = W0[0].T
    w1t, w2t, w3t = W1[0].T, W2[0].T, W3[0].T
    w4t = W4[0].T
    b0c = b0[0][:, None]
    b1c, b2c, b3c = b1[0][:, None], b2[0][:, None], b3[0][:, None]
    b4c = b4[0][:, None]

    full = lambda shape: pl.BlockSpec(shape, lambda i: (0,) * len(shape))
    yT, logits, probs = pl.pallas_call(
        _mlp_body,
        grid=grid,
        in_specs=[
            pl.BlockSpec((in_f, _BLK), lambda i: (0, i)),
            full((hid, in_f)), full((hid, 1)),
            full((hid, hid)), full((hid, 1)),
            full((hid, hid)), full((hid, 1)),
            full((hid, hid)), full((hid, 1)),
            full((out_f, hid)), full((out_f, 1)),
        ],
        out_specs=[
            pl.BlockSpec((out_f, _BLK), lambda i: (0, i)),
            pl.BlockSpec((_BLK, _NUM_MODELS), lambda i: (i, 0)),
            pl.BlockSpec((_BLK, _NUM_MODELS), lambda i: (i, 0)),
        ],
        out_shape=[
            jax.ShapeDtypeStruct((out_f, bsz), jnp.float32),
            jax.ShapeDtypeStruct((bsz, _NUM_MODELS), jnp.float32),
            jax.ShapeDtypeStruct((bsz, _NUM_MODELS), jnp.float32),
        ],
    )(xT, w0t, b0c, w1t, b1c, w2t, b2c, w3t, b3c, w4t, b4c)

    y = yT.T
    model_outputs = jnp.broadcast_to(y[:, None, :], (bsz, n_lods, out_f))
    sel_idx = jnp.zeros((bsz,), jnp.int32)
    return (model_outputs, sel_idx, logits, probs)


# speedup vs baseline: 5.6490x; 2.4889x over previous
import jax
import jax.numpy as jnp
from jax.experimental import pallas as pl

_NUM_MODELS = 64
_BLK = 8192


def _mlp_body(x_ref, w0_ref, b0_ref, w1_ref, b1_ref, w2_ref, b2_ref,
              w3_ref, b3_ref, w4_ref, b4_ref, y_ref):
    x = x_ref[...]
    h = jnp.dot(w0_ref[...], x, preferred_element_type=jnp.float32) + b0_ref[...]
    h = jnp.maximum(h, 0.0)
    h = jnp.dot(w1_ref[...], h, preferred_element_type=jnp.float32) + b1_ref[...]
    h = jnp.maximum(h, 0.0)
    h = jnp.dot(w2_ref[...], h, preferred_element_type=jnp.float32) + b2_ref[...]
    h = jnp.maximum(h, 0.0)
    h = jnp.dot(w3_ref[...], h, preferred_element_type=jnp.float32) + b3_ref[...]
    h = jnp.maximum(h, 0.0)
    y_ref[...] = (jnp.dot(w4_ref[...], h, preferred_element_type=jnp.float32)
                  + b4_ref[...])


def kernel(inputs, lods, W0, b0, W1, b1, W2, b2, W3, b3, W4, b4):
    bsz, in_f = inputs.shape
    hid = W1.shape[-1]
    out_f = W4.shape[-1]
    n_lods = int(lods.shape[0])
    grid = (bsz // _BLK,)

    xT = inputs.T
    w0t = W0[0].T
    w1t, w2t, w3t = W1[0].T, W2[0].T, W3[0].T
    w4t = W4[0].T
    b0c = b0[0][:, None]
    b1c, b2c, b3c = b1[0][:, None], b2[0][:, None], b3[0][:, None]
    b4c = b4[0][:, None]

    full = lambda shape: pl.BlockSpec(shape, lambda i: (0,) * len(shape))
    yT = pl.pallas_call(
        _mlp_body,
        grid=grid,
        in_specs=[
            pl.BlockSpec((in_f, _BLK), lambda i: (0, i)),
            full((hid, in_f)), full((hid, 1)),
            full((hid, hid)), full((hid, 1)),
            full((hid, hid)), full((hid, 1)),
            full((hid, hid)), full((hid, 1)),
            full((out_f, hid)), full((out_f, 1)),
        ],
        out_specs=pl.BlockSpec((out_f, _BLK), lambda i: (0, i)),
        out_shape=jax.ShapeDtypeStruct((out_f, bsz), jnp.float32),
    )(xT, w0t, b0c, w1t, b1c, w2t, b2c, w3t, b3c, w4t, b4c)

    y = yT.T
    model_outputs = jnp.broadcast_to(y[:, None, :], (bsz, n_lods, out_f))
    sel_idx = jnp.zeros((bsz,), jnp.int32)
    logit_row = jnp.concatenate(
        [jnp.zeros((1,), inputs.dtype),
         jnp.full((_NUM_MODELS - 1,), -999.9, inputs.dtype)])
    logits = jnp.broadcast_to(logit_row[None, :], (bsz, _NUM_MODELS))
    prob_row = jnp.concatenate(
        [jnp.ones((1,), inputs.dtype),
         jnp.zeros((_NUM_MODELS - 1,), inputs.dtype)])
    probs = jnp.broadcast_to(prob_row[None, :], (bsz, _NUM_MODELS))
    return (model_outputs, sel_idx, logits, probs)
